# baseline (device time: 524050 ns/iter reference)
import jax
import jax.numpy as jnp
from jax import lax
from jax.experimental import pallas as pl
from jax.experimental.pallas import tpu as pltpu

N_DEV = 32
B, SQ, SKV, DH = 2, 512, 512, 64
H_LOC = 8
DM = 768
DLOC = H_LOC * DH
ROWS = B * SQ
CHUNK = ROWS // N_DEV
MESH = pl.DeviceIdType.MESH


def kernel(x, Wq, K_ext, V_ext, Wo):
    idx = lax.axis_index("i")
    wq_loc = lax.dynamic_slice(Wq, (0, idx * DLOC), (DM, DLOC))
    wo_loc = lax.dynamic_slice(Wo, (idx * DLOC, 0), (DLOC, DM))

    def body(x_ref, wq_ref, k_ref, v_ref, wo_ref, out_ref,
             comm_ref, send_sem, recv_sem, credit_sem):
        my = lax.axis_index("i")
        left = lax.rem(my + N_DEV - 1, N_DEV)
        right = lax.rem(my + 1, N_DEV)

        barrier = pltpu.get_barrier_semaphore()
        pl.semaphore_signal(barrier, inc=1, device_id=(left,),
                            device_id_type=MESH)
        pl.semaphore_signal(barrier, inc=1, device_id=(right,),
                            device_id_type=MESH)
        pl.semaphore_wait(barrier, 2)

        qb = lax.broadcasted_iota(jnp.int32, (SQ, SKV), 0) // 64
        kb = lax.broadcasted_iota(jnp.int32, (SQ, SKV), 1) // 64
        mask = (qb == kb) | (kb == 0) | (((qb + kb) % 3) == 0)

        for b in range(B):
            xb = x_ref[b]
            Qb = jnp.dot(xb, wq_ref[...],
                         preferred_element_type=jnp.float32)
            ctxs = []
            for h in range(H_LOC):
                qh = Qb[:, h * DH:(h + 1) * DH]
                kh = k_ref[b, :, h, :]
                vh = v_ref[b, :, h, :]
                s = lax.dot_general(
                    qh, kh, (((1,), (1,)), ((), ())),
                    preferred_element_type=jnp.float32) * 0.125
                s = jnp.where(mask, s, -1e9)
                m = jnp.max(s, axis=1, keepdims=True)
                w = jnp.exp(s - m)
                w = w / jnp.sum(w, axis=1, keepdims=True)
                ctxs.append(jnp.dot(w, vh,
                                    preferred_element_type=jnp.float32))
            ctx = jnp.concatenate(ctxs, axis=1)
            out_ref[pl.ds(b * SQ, SQ), :] = jnp.dot(
                ctx, wo_ref[...], preferred_element_type=jnp.float32)

        for s in range(2 * (N_DEV - 1)):
            if s < N_DEV - 1:
                send_c = lax.rem(my - s + 2 * N_DEV, N_DEV)
                recv_c = lax.rem(my - s - 1 + 2 * N_DEV, N_DEV)
            else:
                t = s - (N_DEV - 1)
                send_c = lax.rem(my + 1 - t + 2 * N_DEV, N_DEV)
                recv_c = lax.rem(my - t + 2 * N_DEV, N_DEV)

            rdma = pltpu.make_async_remote_copy(
                src_ref=out_ref.at[pl.ds(send_c * CHUNK, CHUNK), :],
                dst_ref=comm_ref,
                send_sem=send_sem,
                recv_sem=recv_sem,
                device_id=(right,),
                device_id_type=MESH,
            )
            rdma.start()
            rdma.wait()

            if s < N_DEV - 1:
                out_ref[pl.ds(recv_c * CHUNK, CHUNK), :] += comm_ref[...]
            else:
                out_ref[pl.ds(recv_c * CHUNK, CHUNK), :] = comm_ref[...]

            pl.semaphore_signal(credit_sem, inc=1, device_id=(left,),
                                device_id_type=MESH)
            pl.semaphore_wait(credit_sem, 1)

    out2d = pl.pallas_call(
        body,
        out_shape=jax.ShapeDtypeStruct((ROWS, DM), jnp.float32),
        in_specs=[pl.BlockSpec(memory_space=pltpu.VMEM)] * 5,
        out_specs=pl.BlockSpec(memory_space=pltpu.VMEM),
        scratch_shapes=[
            pltpu.VMEM((CHUNK, DM), jnp.float32),
            pltpu.SemaphoreType.DMA,
            pltpu.SemaphoreType.DMA,
            pltpu.SemaphoreType.REGULAR,
        ],
        compiler_params=pltpu.CompilerParams(collective_id=0),
    )(x, wq_loc, K_ext, V_ext, wo_loc)
    return out2d.reshape(B, SQ, DM)


# device time: 130951 ns/iter; 4.0019x vs baseline; 4.0019x over previous
import jax
import jax.numpy as jnp
from jax import lax
from jax.experimental import pallas as pl
from jax.experimental.pallas import tpu as pltpu

N_DEV = 32
B, SQ, SKV, DH = 2, 512, 512, 64
H_LOC = 8
DM = 768
DLOC = H_LOC * DH
ROWS = B * SQ
MESH = pl.DeviceIdType.MESH

RS_KS = [1, 2, 4, 8, 16]
AG_KS = [16, 8, 4, 2, 1]
N_STAGES = len(RS_KS) + len(AG_KS)


def kernel(x, Wq, K_ext, V_ext, Wo):
    idx = lax.axis_index("i")
    wq_loc = lax.dynamic_slice(Wq, (0, idx * DLOC), (DM, DLOC))
    wo_loc = lax.dynamic_slice(Wo, (idx * DLOC, 0), (DLOC, DM))

    def body(x_ref, wq_ref, k_ref, v_ref, wo_ref, out_ref,
             comm_ref, send_sem, recv_sem, credit_sems):
        my = lax.axis_index("i")

        qb = lax.broadcasted_iota(jnp.int32, (SQ, SKV), 0) // 64
        kb = lax.broadcasted_iota(jnp.int32, (SQ, SKV), 1) // 64
        mask = (qb == kb) | (kb == 0) | (((qb + kb) % 3) == 0)

        for b in range(B):
            xb = x_ref[b]
            Qb = jnp.dot(xb, wq_ref[...],
                         preferred_element_type=jnp.float32)
            ctxs = []
            for h in range(H_LOC):
                qh = Qb[:, h * DH:(h + 1) * DH]
                kh = k_ref[b, :, h, :]
                vh = v_ref[b, :, h, :]
                s = lax.dot_general(
                    qh, kh, (((1,), (1,)), ((), ())),
                    preferred_element_type=jnp.float32) * 0.125
                s = jnp.where(mask, s, -1e9)
                m = jnp.max(s, axis=1, keepdims=True)
                w = jnp.exp(s - m)
                w = w / jnp.sum(w, axis=1, keepdims=True)
                ctxs.append(jnp.dot(w, vh,
                                    preferred_element_type=jnp.float32))
            ctx = jnp.concatenate(ctxs, axis=1)
            out_ref[pl.ds(b * SQ, SQ), :] = jnp.dot(
                ctx, wo_ref[...], preferred_element_type=jnp.float32)

        p0 = my ^ RS_KS[0]
        barrier = pltpu.get_barrier_semaphore()
        pl.semaphore_signal(barrier, inc=1, device_id=(p0,),
                            device_id_type=MESH)
        pl.semaphore_wait(barrier, 1)

        base = my * 0
        length = ROWS

        for s, k in enumerate(RS_KS):
            partner = my ^ k
            half = length // 2
            bit = (my & k) != 0
            send_base = pl.multiple_of(base + jnp.where(bit, 0, half), 32)
            keep_base = pl.multiple_of(base + jnp.where(bit, half, 0), 32)
            if s > 0:
                pl.semaphore_signal(credit_sems.at[s - 1], inc=1,
                                    device_id=(partner,),
                                    device_id_type=MESH)
                pl.semaphore_wait(credit_sems.at[s - 1], 1)
            rdma = pltpu.make_async_remote_copy(
                src_ref=out_ref.at[pl.ds(send_base, half), :],
                dst_ref=comm_ref.at[pl.ds(0, half), :],
                send_sem=send_sem,
                recv_sem=recv_sem,
                device_id=(partner,),
                device_id_type=MESH,
            )
            rdma.start()
            rdma.wait()
            out_ref[pl.ds(keep_base, half), :] += comm_ref[pl.ds(0, half), :]
            base = keep_base
            length = half

        for s, k in enumerate(AG_KS):
            stage = len(RS_KS) + s
            partner = my ^ k
            bit = (my & k) != 0
            partner_base = base + jnp.where(bit, -length, length)
            pl.semaphore_signal(credit_sems.at[stage - 1], inc=1,
                                device_id=(partner,),
                                device_id_type=MESH)
            pl.semaphore_wait(credit_sems.at[stage - 1], 1)
            ag_base = pl.multiple_of(base, 32)
            rdma = pltpu.make_async_remote_copy(
                src_ref=out_ref.at[pl.ds(ag_base, length), :],
                dst_ref=out_ref.at[pl.ds(ag_base, length), :],
                send_sem=send_sem,
                recv_sem=recv_sem,
                device_id=(partner,),
                device_id_type=MESH,
            )
            rdma.start()
            rdma.wait()
            base = jnp.where(bit, base - length, base)
            length = length * 2

    out2d = pl.pallas_call(
        body,
        out_shape=jax.ShapeDtypeStruct((ROWS, DM), jnp.float32),
        in_specs=[pl.BlockSpec(memory_space=pltpu.VMEM)] * 5,
        out_specs=pl.BlockSpec(memory_space=pltpu.VMEM),
        scratch_shapes=[
            pltpu.VMEM((ROWS // 2, DM), jnp.float32),
            pltpu.SemaphoreType.DMA,
            pltpu.SemaphoreType.DMA,
            pltpu.SemaphoreType.REGULAR((N_STAGES - 1,)),
        ],
        compiler_params=pltpu.CompilerParams(collective_id=0),
    )(x, wq_loc, K_ext, V_ext, wo_loc)
    return out2d.reshape(B, SQ, DM)
